# baseline (speedup 1.0000x reference)
import functools

import jax
import jax.numpy as jnp
from jax import lax
from jax.experimental import pallas as pl
from jax.experimental.pallas import tpu as pltpu
from jax.experimental.pallas import tpu_sc as plsc

_N_EDGES = 6400000
_N_NODES = 100000
_NC = 2
_NS = 16
_NT = _NC * _NS
_EPT = _N_EDGES // _NT
_CHUNK = 10000
_NCHUNK = _EPT // _CHUNK
_VPC = _CHUNK // 16
_ACC = 100352
_ACCP = _ACC + 2048
_SENT = _N_NODES
_G = 128


def _sc_partial_counts(edges):
    mesh = plsc.VectorSubcoreMesh(core_axis_name="c", subcore_axis_name="s")

    @functools.partial(
        pl.kernel,
        out_type=jax.ShapeDtypeStruct((_NC, _ACC), jnp.float32),
        mesh=mesh,
        scratch_types=[
            pltpu.VMEM((_ACCP,), jnp.float32),
            pltpu.VMEM((2 * (_CHUNK + 16),), jnp.int32),
            pltpu.VMEM((_G,), jnp.int32),
            pltpu.VMEM_SHARED((_ACCP,), jnp.float32),
            pltpu.SemaphoreType.DMA((2,)),
        ],
        compiler_params=pltpu.CompilerParams(needs_layout_passes=False),
    )
    def body(edges_hbm, part_hbm, acc, ebuf, idxb, shared, sem):
        cid = lax.axis_index("c")
        sid = lax.axis_index("s")
        wid = cid * _NS + sid
        base = wid * _EPT

        zero16 = jnp.zeros((16,), jnp.float32)
        iota1 = lax.iota(jnp.int32, 16) + 1
        sentv = jnp.full((16,), _SENT, jnp.int32)
        is_tail_tile = wid == _NT - 1

        def dma_start(g):
            sbase = pl.multiple_of(lax.rem(g, 2) * (_CHUNK + 16), 8)
            off = pl.multiple_of(base + g * _CHUNK, 8)
            short_cond = jnp.logical_and(g == _NCHUNK - 1, is_tail_tile)

            @pl.when(jnp.logical_not(short_cond))
            def _():
                pltpu.make_async_copy(
                    edges_hbm.at[pl.ds(off, _CHUNK + 16)],
                    ebuf.at[pl.ds(sbase, _CHUNK + 16)],
                    sem.at[lax.rem(g, 2)],
                ).start()

            @pl.when(short_cond)
            def _():
                pltpu.make_async_copy(
                    edges_hbm.at[pl.ds(off, _CHUNK)],
                    ebuf.at[pl.ds(sbase, _CHUNK)],
                    sem.at[lax.rem(g, 2)],
                ).start()

        def dma_wait(g):
            sbase = pl.multiple_of(lax.rem(g, 2) * (_CHUNK + 16), 8)
            off = pl.multiple_of(base + g * _CHUNK, 8)
            short_cond = jnp.logical_and(g == _NCHUNK - 1, is_tail_tile)

            @pl.when(jnp.logical_not(short_cond))
            def _():
                pltpu.make_async_copy(
                    edges_hbm.at[pl.ds(off, _CHUNK + 16)],
                    ebuf.at[pl.ds(sbase, _CHUNK + 16)],
                    sem.at[lax.rem(g, 2)],
                ).wait()

            @pl.when(short_cond)
            def _():
                pltpu.make_async_copy(
                    edges_hbm.at[pl.ds(off, _CHUNK)],
                    ebuf.at[pl.ds(sbase, _CHUNK)],
                    sem.at[lax.rem(g, 2)],
                ).wait()
                ebuf[pl.ds(sbase + _CHUNK, 16)] = sentv

        _U = 20
        dma_start(0)

        def zbody(i, _):
            for u in range(16):
                acc[pl.ds(i * 256 + u * 16, 16)] = zero16
            return 0

        lax.fori_loop(0, _ACCP // 256, zbody, 0)

        zs = _ACCP // _NS
        pltpu.sync_copy(
            acc.at[pl.ds(sid * zs, zs)], shared.at[pl.ds(sid * zs, zs)]
        )

        def chunk_body(g, wlo):
            @pl.when(g + 1 < _NCHUNK)
            def _():
                dma_start(g + 1)

            dma_wait(g)
            sbase = lax.rem(g, 2) * (_CHUNK + 16)
            off = base + g * _CHUNK

            @plsc.parallel_loop(0, _VPC, unroll=_U)
            def _(k):
                idx = sbase + k * 16
                cur = ebuf[pl.ds(idx, 16)]
                nxt = ebuf[pl.ds(idx + 1, 16)]
                m = cur != nxt
                pos = (iota1 + (off - sbase + idx)).astype(jnp.float32)
                plsc.addupdate_scatter(acc, [cur], pos, mask=m)
                plsc.addupdate_scatter(acc, [nxt], -pos, mask=m)

            first = ebuf[pl.ds(sbase, 16)][0]
            return jnp.where(g == 0, first, wlo)

        wlo = lax.fori_loop(0, _NCHUNK, chunk_body, jnp.int32(0))

        whi = ebuf[pl.ds(((_NCHUNK - 1) % 2) * (_CHUNK + 16) + _CHUNK, 16)][0]
        w0 = pl.multiple_of((wlo // _G) * _G, 8)
        ngran = (whi - w0) // _G + 1

        plsc.subcore_barrier()

        iota16 = lax.iota(jnp.int32, 16)

        def merge_body(i, _):
            gbase = pl.multiple_of(w0 + i * _G, 8)
            for u in range(_G // 16):
                idxb[pl.ds(u * 16, 16)] = iota16 + (gbase + u * 16)
            pltpu.sync_copy(
                acc.at[pl.ds(gbase, _G)], shared.at[idxb], add=True
            )
            return 0

        lax.fori_loop(0, ngran, merge_body, 0)

        plsc.subcore_barrier()

        cs = _ACC // _NS
        pltpu.sync_copy(
            shared.at[pl.ds(sid * cs, cs)],
            part_hbm.at[cid, pl.ds(sid * cs, cs)],
        )

    return body(edges)


def _combine(part, a3, cet):
    rows = _ACC // 128
    br = 112

    def body(p_ref, a_ref, c_ref, o_ref):
        s = jnp.sum(p_ref[...], axis=0)
        o_ref[...] = a_ref[...] / (c_ref[...] + s)

    return pl.pallas_call(
        body,
        grid=(rows // br,),
        in_specs=[
            pl.BlockSpec((_NC, br, 128), lambda i: (0, i, 0)),
            pl.BlockSpec((br, 128), lambda i: (i, 0)),
            pl.BlockSpec((br, 128), lambda i: (i, 0)),
        ],
        out_specs=pl.BlockSpec((br, 128), lambda i: (i, 0)),
        out_shape=jax.ShapeDtypeStruct((rows, 128), jnp.float32),
    )(part, a3, cet)


def kernel(arg0_1, arg3_1, convert_element_type, convert_element_type_1):
    del convert_element_type_1
    edges = arg0_1.astype(jnp.int32)
    part = _sc_partial_counts(edges)
    rows = _ACC // 128
    a3 = jnp.pad(arg3_1, (0, _ACC - _N_NODES)).reshape(rows, 128)
    cet = jnp.pad(convert_element_type, (0, _ACC - _N_NODES)).reshape(rows, 128)
    out = _combine(part.reshape(_NC, rows, 128), a3, cet)
    return out.reshape(_ACC)[:_N_NODES]


# speedup vs baseline: 1.1100x; 1.1100x over previous
import functools

import jax
import jax.numpy as jnp
from jax import lax
from jax.experimental import pallas as pl
from jax.experimental.pallas import tpu as pltpu
from jax.experimental.pallas import tpu_sc as plsc

_N_EDGES = 6400000
_N_NODES = 100000
_NC = 2
_NS = 16
_NT = _NC * _NS
_EPT = _N_EDGES // _NT
_CHUNK = 10000
_NCHUNK = _EPT // _CHUNK
_VPC = _CHUNK // 16
_ACC = 100352
_ACCP = _ACC + 2048
_SENT = _N_NODES
_G = 128


def _sc_partial_counts(edges):
    mesh = plsc.VectorSubcoreMesh(core_axis_name="c", subcore_axis_name="s")

    @functools.partial(
        pl.kernel,
        out_type=jax.ShapeDtypeStruct((_NC, _ACC), jnp.float32),
        mesh=mesh,
        scratch_types=[
            pltpu.VMEM((_ACCP,), jnp.float32),
            pltpu.VMEM((2 * (_CHUNK + 16),), jnp.int32),
            pltpu.VMEM((_G,), jnp.int32),
            pltpu.VMEM((1600,), jnp.float32),
            pltpu.VMEM_SHARED((_ACCP,), jnp.float32),
            pltpu.SemaphoreType.DMA((2,)),
        ],
        compiler_params=pltpu.CompilerParams(needs_layout_passes=False),
    )
    def body(edges_hbm, part_hbm, acc, ebuf, idxb, zbuf, shared, sem):
        cid = lax.axis_index("c")
        sid = lax.axis_index("s")
        wid = cid * _NS + sid
        base = wid * _EPT

        zero16 = jnp.zeros((16,), jnp.float32)
        iota1 = lax.iota(jnp.int32, 16) + 1
        sentv = jnp.full((16,), _SENT, jnp.int32)
        is_tail_tile = wid == _NT - 1

        def dma_start(g):
            sbase = pl.multiple_of(lax.rem(g, 2) * (_CHUNK + 16), 8)
            off = pl.multiple_of(base + g * _CHUNK, 8)
            short_cond = jnp.logical_and(g == _NCHUNK - 1, is_tail_tile)

            @pl.when(jnp.logical_not(short_cond))
            def _():
                pltpu.make_async_copy(
                    edges_hbm.at[pl.ds(off, _CHUNK + 16)],
                    ebuf.at[pl.ds(sbase, _CHUNK + 16)],
                    sem.at[lax.rem(g, 2)],
                ).start()

            @pl.when(short_cond)
            def _():
                pltpu.make_async_copy(
                    edges_hbm.at[pl.ds(off, _CHUNK)],
                    ebuf.at[pl.ds(sbase, _CHUNK)],
                    sem.at[lax.rem(g, 2)],
                ).start()

        def dma_wait(g):
            sbase = pl.multiple_of(lax.rem(g, 2) * (_CHUNK + 16), 8)
            off = pl.multiple_of(base + g * _CHUNK, 8)
            short_cond = jnp.logical_and(g == _NCHUNK - 1, is_tail_tile)

            @pl.when(jnp.logical_not(short_cond))
            def _():
                pltpu.make_async_copy(
                    edges_hbm.at[pl.ds(off, _CHUNK + 16)],
                    ebuf.at[pl.ds(sbase, _CHUNK + 16)],
                    sem.at[lax.rem(g, 2)],
                ).wait()

            @pl.when(short_cond)
            def _():
                pltpu.make_async_copy(
                    edges_hbm.at[pl.ds(off, _CHUNK)],
                    ebuf.at[pl.ds(sbase, _CHUNK)],
                    sem.at[lax.rem(g, 2)],
                ).wait()
                ebuf[pl.ds(sbase + _CHUNK, 16)] = sentv

        _U = 10
        dma_start(0)

        zs = _ACCP // _NS

        def zbody(i, _):
            for u in range(10):
                zbuf[pl.ds(i * 160 + u * 16, 16)] = zero16
            return 0

        lax.fori_loop(0, 10, zbody, 0)

        for q in range(4):
            pltpu.sync_copy(
                zbuf, shared.at[pl.ds(sid * zs + q * 1600, 1600)]
            )

        def chunk_body(g, carry):
            wlo, zptr = carry

            @pl.when(g + 1 < _NCHUNK)
            def _():
                dma_start(g + 1)

            dma_wait(g)
            sbase = lax.rem(g, 2) * (_CHUNK + 16)
            off = base + g * _CHUNK

            first = ebuf[pl.ds(sbase, 16)][0]
            wlo = jnp.where(g == 0, first, wlo)
            zptr = jnp.where(g == 0, (wlo // _G) * _G, zptr)
            zhi = ebuf[pl.ds(sbase + _CHUNK, 16)][0]
            nz = (zhi - zptr + 16) // 16

            def zrun(i, _):
                acc[pl.ds(pl.multiple_of(zptr + i * 16, 8), 16)] = zero16
                return 0

            lax.fori_loop(0, nz, zrun, 0)
            zptr = zptr + nz * 16

            @plsc.parallel_loop(0, _VPC, unroll=_U)
            def _(k):
                idx = sbase + k * 16
                cur = ebuf[pl.ds(idx, 16)]
                nxt = ebuf[pl.ds(idx + 1, 16)]
                m = cur != nxt
                pos = (iota1 + (off - sbase + idx)).astype(jnp.float32)
                plsc.addupdate_scatter(acc, [cur], pos, mask=m)
                plsc.addupdate_scatter(acc, [nxt], -pos, mask=m)

            return (wlo, zptr)

        wlo, zptr = lax.fori_loop(
            0, _NCHUNK, chunk_body, (jnp.int32(0), jnp.int32(0))
        )

        for i in range(10):
            acc[pl.ds(pl.multiple_of(zptr + i * 16, 8), 16)] = zero16

        whi = ebuf[pl.ds(((_NCHUNK - 1) % 2) * (_CHUNK + 16) + _CHUNK, 16)][0]
        w0 = pl.multiple_of((wlo // _G) * _G, 8)
        ngran = (whi - w0) // _G + 1

        plsc.subcore_barrier()

        iota16 = lax.iota(jnp.int32, 16)

        def merge_body(i, _):
            gbase = pl.multiple_of(w0 + i * _G, 8)
            for u in range(_G // 16):
                idxb[pl.ds(u * 16, 16)] = iota16 + (gbase + u * 16)
            pltpu.sync_copy(
                acc.at[pl.ds(gbase, _G)], shared.at[idxb], add=True
            )
            return 0

        lax.fori_loop(0, ngran, merge_body, 0)

        plsc.subcore_barrier()

        cs = _ACC // _NS
        pltpu.sync_copy(
            shared.at[pl.ds(sid * cs, cs)],
            part_hbm.at[cid, pl.ds(sid * cs, cs)],
        )

    return body(edges)


def _combine(part, a3, cet):
    rows = _ACC // 128
    br = 112

    def body(p_ref, a_ref, c_ref, o_ref):
        s = jnp.sum(p_ref[...], axis=0)
        o_ref[...] = a_ref[...] / (c_ref[...] + s)

    return pl.pallas_call(
        body,
        grid=(rows // br,),
        in_specs=[
            pl.BlockSpec((_NC, br, 128), lambda i: (0, i, 0)),
            pl.BlockSpec((br, 128), lambda i: (i, 0)),
            pl.BlockSpec((br, 128), lambda i: (i, 0)),
        ],
        out_specs=pl.BlockSpec((br, 128), lambda i: (i, 0)),
        out_shape=jax.ShapeDtypeStruct((rows, 128), jnp.float32),
    )(part, a3, cet)


def kernel(arg0_1, arg3_1, convert_element_type, convert_element_type_1):
    del convert_element_type_1
    edges = arg0_1.astype(jnp.int32)
    part = _sc_partial_counts(edges)
    rows = _ACC // 128
    a3 = jnp.pad(arg3_1, (0, _ACC - _N_NODES)).reshape(rows, 128)
    cet = jnp.pad(convert_element_type, (0, _ACC - _N_NODES)).reshape(rows, 128)
    out = _combine(part.reshape(_NC, rows, 128), a3, cet)
    return out.reshape(_ACC)[:_N_NODES]
